# Initial kernel scaffold; baseline (speedup 1.0000x reference)
import jax
import jax.numpy as jnp
from jax.experimental import pallas as pl


def kernel(x, edge_index, edge_type, lin_w, lin_b, comp1, bases1, root1, bias1, comp2, bases2, root2, bias2):
    raise NotImplementedError("write your pallas kernel here")


# speedup vs baseline: 16.6230x; 16.6230x over previous
import functools

import jax
import jax.numpy as jnp
from jax import lax
from jax.experimental import pallas as pl
from jax.experimental.pallas import tpu as pltpu
from jax.experimental.pallas import tpu_sc as plsc

_N = 10000
_E = 320000
_R = 8
_D = 128
_NR = _N * _R
_NC = 2
_NS = 16
_NW = _NC * _NS
_EPW = _E // _NW
_EPT = _E // _NS
_DH2 = _D // 2
_CH = 80
_NCHUNK = _EPT // _CH
_CCH = 2000
_CPW = _NR // _NS
_RPT = _N // _NS

_mesh = plsc.VectorSubcoreMesh(core_axis_name="c", subcore_axis_name="s")
_sc_params = pltpu.CompilerParams(needs_layout_passes=False,
                                  use_tc_tiling_on_sc=False)


def _mm_bias(a, b, bias, bm, bn):
    M, K = a.shape
    Nn = b.shape[1]

    def body(a_ref, b_ref, s_ref, o_ref):
        o_ref[...] = jnp.dot(a_ref[...], b_ref[...],
                             preferred_element_type=jnp.float32) + s_ref[...]

    return pl.pallas_call(
        body,
        grid=(M // bm, Nn // bn),
        in_specs=[
            pl.BlockSpec((bm, K), lambda i, j: (i, 0)),
            pl.BlockSpec((K, bn), lambda i, j: (0, j)),
            pl.BlockSpec((1, bn), lambda i, j: (0, j)),
        ],
        out_specs=pl.BlockSpec((bm, bn), lambda i, j: (i, j)),
        out_shape=jax.ShapeDtypeStruct((M, Nn), jnp.float32),
    )(a, b, bias)


def _combine(hroot, parts):
    M, Nn = hroot.shape

    def body(h_ref, p_ref, o_ref):
        o_ref[...] = h_ref[...] + jnp.concatenate(
            [p_ref[0], p_ref[1]], axis=1)

    bm = 1000
    return pl.pallas_call(
        body,
        grid=(M // bm,),
        in_specs=[
            pl.BlockSpec((bm, Nn), lambda i: (i, 0)),
            pl.BlockSpec((_NC, bm, Nn // 2), lambda i: (0, i, 0)),
        ],
        out_specs=pl.BlockSpec((bm, Nn), lambda i: (i, 0)),
        out_shape=jax.ShapeDtypeStruct((M, Nn), jnp.float32),
    )(hroot, parts)


def _invc(cparts):
    def body(c_ref, o_ref):
        o_ref[...] = 1.0 / jnp.maximum(c_ref[0] + c_ref[1], 1.0)

    return pl.pallas_call(
        body,
        out_shape=jax.ShapeDtypeStruct((625, 128), jnp.float32),
    )(cparts)


@functools.partial(
    pl.kernel,
    out_type=jax.ShapeDtypeStruct((_NC * _NR,), jnp.float32),
    mesh=_mesh,
    scratch_types=[
        pltpu.VMEM_SHARED((_NR,), jnp.float32),
        pltpu.VMEM((_CCH,), jnp.int32),
        pltpu.VMEM((_CCH,), jnp.int32),
        pltpu.VMEM((_CCH,), jnp.int32),
        pltpu.VMEM((_CCH,), jnp.float32),
        pltpu.VMEM((_CPW,), jnp.float32),
    ],
    compiler_params=_sc_params,
)
def _counts_kernel(dst_hbm, et_hbm, zero_hbm, out_hbm,
                   cacc, dbuf, ebuf, fbuf, obuf, stage):
    cid = lax.axis_index("c")
    sid = lax.axis_index("s")
    wid = sid * _NC + cid
    pltpu.sync_copy(zero_hbm.at[pl.ds(sid * _CPW, _CPW)], stage)
    pltpu.sync_copy(stage, cacc.at[pl.ds(sid * _CPW, _CPW)])
    def fill(g, _):
        obuf[pl.ds(g * 16, 16)] = jnp.ones((16,), jnp.float32)
        return 0
    lax.fori_loop(0, _CCH // 16, fill, 0)
    plsc.subcore_barrier()

    base = wid * _EPW

    def chunk(k, _):
        off = base + k * _CCH
        pltpu.sync_copy(dst_hbm.at[pl.ds(off, _CCH)], dbuf)
        pltpu.sync_copy(et_hbm.at[pl.ds(off, _CCH)], ebuf)

        def grp(g, _):
            sl = pl.ds(g * 16, 16)
            fbuf[sl] = dbuf[sl] * 8 + ebuf[sl]
            return 0
        lax.fori_loop(0, _CCH // 16, grp, 0)
        pltpu.sync_copy(obuf, cacc.at[fbuf], add=True)
        return 0
    lax.fori_loop(0, _EPW // _CCH, chunk, 0)

    plsc.subcore_barrier()
    pltpu.sync_copy(cacc.at[pl.ds(sid * _CPW, _CPW)], stage)
    pltpu.sync_copy(stage, out_hbm.at[pl.ds(cid * _NR + sid * _CPW, _CPW)])


@functools.partial(
    pl.kernel,
    out_type=jax.ShapeDtypeStruct((_NC, _N, _DH2), jnp.float32),
    mesh=_mesh,
    scratch_types=[
        pltpu.VMEM_SHARED((_N, _DH2), jnp.float32),
        pltpu.VMEM((_CH,), jnp.int32),
        pltpu.VMEM((_CH,), jnp.int32),
        pltpu.VMEM((_CH,), jnp.int32),
        pltpu.VMEM((_CH,), jnp.int32),
        pltpu.VMEM((_CH,), jnp.int32),
        pltpu.VMEM((_CH,), jnp.float32),
        pltpu.VMEM((_CH, _DH2), jnp.float32),
        pltpu.VMEM((16,), jnp.float32),
        pltpu.VMEM((16, _DH2), jnp.float32),
        pltpu.SemaphoreType.DMA,
        pltpu.SemaphoreType.DMA,
    ],
    compiler_params=_sc_params,
)
def _edge_kernel(hcat_hbm, src_hbm, dst_hbm, et_hbm, invc_hbm,
                 out_hbm, acc, sbuf, dbuf, ebuf, gbuf, fbuf, ibuf, rows,
                 iscr, stage, sem, sem2):
    cid = lax.axis_index("c")
    sid = lax.axis_index("s")
    for a in range(16):
        for b in range(_DH2 // 16):
            stage[a, pl.ds(b * 16, 16)] = jnp.zeros((16,), jnp.float32)
    nchunks = _N // 16
    nit = (nchunks + _NS - 1) // _NS

    def zslice(t, _):
        idx = t * _NS + sid

        @pl.when(idx < nchunks)
        def _():
            pltpu.sync_copy(stage, acc.at[pl.ds(idx * 16, 16), :])
        return 0
    lax.fori_loop(0, nit, zslice, 0)
    plsc.subcore_barrier()

    base = sid * _EPT

    def chunk(k, _):
        off = base + k * _CH
        pltpu.sync_copy(src_hbm.at[pl.ds(off, _CH)], sbuf)
        pltpu.sync_copy(dst_hbm.at[pl.ds(off, _CH)], dbuf)
        pltpu.sync_copy(et_hbm.at[pl.ds(off, _CH)], ebuf)
        for g in range(_CH // 16):
            sl = pl.ds(g * 16, 16)
            e16 = ebuf[sl]
            gbuf[sl] = (sbuf[sl] * 8 + e16) * 2 + cid
            fbuf[sl] = dbuf[sl] * 8 + e16
        ic = pltpu.async_copy(invc_hbm.at[fbuf], ibuf, sem2)
        rc = pltpu.async_copy(hcat_hbm.at[gbuf], rows, sem)
        ic.wait()
        rc.wait()
        for g in range(_CH // 16):
            v16 = ibuf[pl.ds(g * 16, 16)]
            for i in range(16):
                spl = jnp.broadcast_to(v16[i], (16,))
                r = g * 16 + i
                for j in range(_DH2 // 16):
                    cs = pl.ds(j * 16, 16)
                    rows[r, cs] = rows[r, cs] * spl
        pltpu.sync_copy(rows, acc.at[dbuf], add=True)
        return 0
    lax.fori_loop(0, _NCHUNK, chunk, 0)

    plsc.subcore_barrier()

    def oslice(t, _):
        idx = t * _NS + sid

        @pl.when(idx < nchunks)
        def _():
            pltpu.sync_copy(acc.at[pl.ds(idx * 16, 16), :], stage)
            pltpu.sync_copy(stage, out_hbm.at[cid, pl.ds(idx * 16, 16), :])
        return 0
    lax.fori_loop(0, nit, oslice, 0)


def _layer(h, src, dst, et, invc_flat, comp, bases, root, bias):
    nb = comp.shape[1]
    comp_p = jnp.pad(comp, ((0, 0), (0, 32 - nb)))
    bases_p = jnp.pad(bases.reshape(nb, -1), ((0, 32 - nb), (0, 0)))
    wflat = _mm_bias(comp_p, bases_p, jnp.zeros((1, _D * _D), jnp.float32),
                     _R, 2048)
    wcat = wflat.reshape(_R, _D, _D).transpose(1, 0, 2).reshape(_D, _R * _D)
    zcat = jnp.zeros((1, _R * _D), jnp.float32)
    hcat = _mm_bias(h, wcat, zcat, 1000, _R * _D).reshape(2 * _NR, _DH2)
    hroot = _mm_bias(h, root, bias[None, :], 1000, _D)
    parts = _edge_kernel(hcat, src, dst, et, invc_flat)
    return _combine(hroot, parts)


def kernel(x, edge_index, edge_type, lin_w, lin_b,
           comp1, bases1, root1, bias1, comp2, bases2, root2, bias2):
    src = edge_index[0]
    dst = edge_index[1]
    et = edge_type
    zc = jnp.zeros((_NR,), jnp.float32)

    h0 = _mm_bias(x, lin_w.T, lin_b[None, :], 1000, _D)

    cparts = _counts_kernel(dst, et, zc)
    invc_flat = _invc(cparts.reshape(_NC, 625, _D)).reshape(_NR)

    comp_s = jnp.stack([comp1, comp2])
    bases_s = jnp.stack([bases1, bases2])
    root_s = jnp.stack([root1, root2])
    bias_s = jnp.stack([bias1, bias2])

    def body(h, ws):
        comp, bases, root, bias = ws
        return _layer(h, src, dst, et, invc_flat, comp, bases, root, bias), 0

    h2, _ = lax.scan(body, h0, (comp_s, bases_s, root_s, bias_s))
    return h2


# speedup vs baseline: 36.5723x; 2.2001x over previous
import functools

import jax
import jax.numpy as jnp
from jax import lax
from jax.experimental import pallas as pl
from jax.experimental.pallas import tpu as pltpu
from jax.experimental.pallas import tpu_sc as plsc

_N = 10000
_E = 320000
_R = 8
_D = 128
_NR = _N * _R
_NC = 2
_NS = 16
_NW = _NC * _NS
_EPW = _E // _NW
_EPT = _E // _NS
_DH2 = _D // 2
_CH = 80
_NCHUNK = _EPT // _CH
_CCH = 2000
_CPW = _NR // _NS
_RPT = _N // _NS

_mesh = plsc.VectorSubcoreMesh(core_axis_name="c", subcore_axis_name="s")
_sc_params = pltpu.CompilerParams(needs_layout_passes=False,
                                  use_tc_tiling_on_sc=False)


def _mm_bias(a, b, bias, bm, bn):
    M, K = a.shape
    Nn = b.shape[1]

    def body(a_ref, b_ref, s_ref, o_ref):
        o_ref[...] = jnp.dot(a_ref[...], b_ref[...],
                             preferred_element_type=jnp.float32) + s_ref[...]

    return pl.pallas_call(
        body,
        grid=(M // bm, Nn // bn),
        in_specs=[
            pl.BlockSpec((bm, K), lambda i, j: (i, 0)),
            pl.BlockSpec((K, bn), lambda i, j: (0, j)),
            pl.BlockSpec((1, bn), lambda i, j: (0, j)),
        ],
        out_specs=pl.BlockSpec((bm, bn), lambda i, j: (i, j)),
        out_shape=jax.ShapeDtypeStruct((M, Nn), jnp.float32),
    )(a, b, bias)


def _combine(hroot, parts):
    M, Nn = hroot.shape

    def body(h_ref, p_ref, o_ref):
        o_ref[...] = h_ref[...] + jnp.concatenate(
            [p_ref[0], p_ref[1]], axis=1)

    bm = 1000
    return pl.pallas_call(
        body,
        grid=(M // bm,),
        in_specs=[
            pl.BlockSpec((bm, Nn), lambda i: (i, 0)),
            pl.BlockSpec((_NC, bm, Nn // 2), lambda i: (0, i, 0)),
        ],
        out_specs=pl.BlockSpec((bm, Nn), lambda i: (i, 0)),
        out_shape=jax.ShapeDtypeStruct((M, Nn), jnp.float32),
    )(hroot, parts)


def _invc(cparts):
    def body(c_ref, o_ref):
        o_ref[...] = 1.0 / jnp.maximum(c_ref[0] + c_ref[1], 1.0)

    return pl.pallas_call(
        body,
        out_shape=jax.ShapeDtypeStruct((625, 128), jnp.float32),
    )(cparts)


@functools.partial(
    pl.kernel,
    out_type=jax.ShapeDtypeStruct((_NC * _NR,), jnp.float32),
    mesh=_mesh,
    scratch_types=[
        pltpu.VMEM_SHARED((_NR,), jnp.float32),
        pltpu.VMEM((_CCH,), jnp.int32),
        pltpu.VMEM((_CCH,), jnp.int32),
        pltpu.VMEM((_CCH,), jnp.int32),
        pltpu.VMEM((_CCH,), jnp.float32),
        pltpu.VMEM((_CPW,), jnp.float32),
    ],
    compiler_params=_sc_params,
)
def _counts_kernel(dst_hbm, et_hbm, zero_hbm, out_hbm,
                   cacc, dbuf, ebuf, fbuf, obuf, stage):
    cid = lax.axis_index("c")
    sid = lax.axis_index("s")
    wid = sid * _NC + cid
    pltpu.sync_copy(zero_hbm.at[pl.ds(sid * _CPW, _CPW)], stage)
    pltpu.sync_copy(stage, cacc.at[pl.ds(sid * _CPW, _CPW)])
    def fill(g, _):
        obuf[pl.ds(g * 16, 16)] = jnp.ones((16,), jnp.float32)
        return 0
    lax.fori_loop(0, _CCH // 16, fill, 0)
    plsc.subcore_barrier()

    base = wid * _EPW

    def chunk(k, _):
        off = base + k * _CCH
        pltpu.sync_copy(dst_hbm.at[pl.ds(off, _CCH)], dbuf)
        pltpu.sync_copy(et_hbm.at[pl.ds(off, _CCH)], ebuf)

        def grp(g, _):
            sl = pl.ds(g * 16, 16)
            fbuf[sl] = dbuf[sl] * 8 + ebuf[sl]
            return 0
        lax.fori_loop(0, _CCH // 16, grp, 0)
        pltpu.sync_copy(obuf, cacc.at[fbuf], add=True)
        return 0
    lax.fori_loop(0, _EPW // _CCH, chunk, 0)

    plsc.subcore_barrier()
    pltpu.sync_copy(cacc.at[pl.ds(sid * _CPW, _CPW)], stage)
    pltpu.sync_copy(stage, out_hbm.at[pl.ds(cid * _NR + sid * _CPW, _CPW)])


@functools.partial(
    pl.kernel,
    out_type=jax.ShapeDtypeStruct((_NC, _N, _DH2), jnp.float32),
    mesh=_mesh,
    scratch_types=[
        pltpu.VMEM_SHARED((_N, _DH2), jnp.float32),
        pltpu.VMEM((_EPT,), jnp.int32),
        pltpu.VMEM((_EPT,), jnp.int32),
        pltpu.VMEM((_EPT,), jnp.int32),
        pltpu.VMEM((_CH, _DH2), jnp.float32),
        pltpu.VMEM((_CH, _DH2), jnp.float32),
        pltpu.VMEM((_CH,), jnp.float32),
        pltpu.VMEM((_CH,), jnp.float32),
        pltpu.VMEM((_CH,), jnp.int32),
        pltpu.VMEM((_CH,), jnp.int32),
        pltpu.VMEM((16, _DH2), jnp.float32),
        pltpu.SemaphoreType.DMA,
        pltpu.SemaphoreType.DMA,
        pltpu.SemaphoreType.DMA,
        pltpu.SemaphoreType.DMA,
        pltpu.SemaphoreType.DMA,
    ],
    compiler_params=_sc_params,
)
def _edge_kernel(hcat_hbm, src_hbm, dst_hbm, et_hbm, invc_hbm,
                 out_hbm, acc, gbuf, fbuf, dbuf, rows0, rows1, ibuf0, ibuf1,
                 didx0, didx1, stage, semg0, semg1, semi0, semi1, semx):
    cid = lax.axis_index("c")
    sid = lax.axis_index("s")
    for a in range(16):
        for b in range(_DH2 // 16):
            stage[a, pl.ds(b * 16, 16)] = jnp.zeros((16,), jnp.float32)
    nchunks = _N // 16
    nit = (nchunks + _NS - 1) // _NS

    def zslice(t, _):
        idx = t * _NS + sid

        @pl.when(idx < nchunks)
        def _():
            pltpu.sync_copy(stage, acc.at[pl.ds(idx * 16, 16), :])
        return 0
    lax.fori_loop(0, nit, zslice, 0)
    plsc.subcore_barrier()

    base = sid * _EPT

    c1 = pltpu.async_copy(src_hbm.at[pl.ds(base, _EPT)], gbuf, semg0)
    c2 = pltpu.async_copy(et_hbm.at[pl.ds(base, _EPT)], fbuf, semg1)
    c3 = pltpu.async_copy(dst_hbm.at[pl.ds(base, _EPT)], dbuf, semi0)
    c1.wait()
    c2.wait()
    c3.wait()

    def idxgrp(g, _):
        sl = pl.ds(g * 16, 16)
        e16 = fbuf[sl]
        gbuf[sl] = (gbuf[sl] * 8 + e16) * 2 + cid
        fbuf[sl] = dbuf[sl] * 8 + e16
        return 0
    lax.fori_loop(0, _EPT // 16, idxgrp, 0)

    pltpu.async_copy(hcat_hbm.at[gbuf.at[pl.ds(0, _CH)]], rows0, semg0)
    pltpu.async_copy(invc_hbm.at[fbuf.at[pl.ds(0, _CH)]], ibuf0, semi0)

    def process(k, rows_c, ibuf_c, didx_c, semg_c, semi_c,
                rows_n, ibuf_n, semg_n, semi_n):
        pltpu.make_async_copy(
            hcat_hbm.at[gbuf.at[pl.ds(0, _CH)]], rows_c, semg_c).wait()
        pltpu.make_async_copy(
            invc_hbm.at[fbuf.at[pl.ds(0, _CH)]], ibuf_c, semi_c).wait()

        @pl.when(k + 1 < _NCHUNK)
        def _():
            off = (k + 1) * _CH
            pltpu.async_copy(
                hcat_hbm.at[gbuf.at[pl.ds(off, _CH)]], rows_n, semg_n)
            pltpu.async_copy(
                invc_hbm.at[fbuf.at[pl.ds(off, _CH)]], ibuf_n, semi_n)
        off0 = k * _CH
        for g in range(_CH // 16):
            didx_c[pl.ds(g * 16, 16)] = dbuf[pl.ds(off0 + g * 16, 16)]
        for g in range(_CH // 16):
            v16 = ibuf_c[pl.ds(g * 16, 16)]
            for i in range(16):
                spl = jnp.broadcast_to(v16[i], (16,))
                r = g * 16 + i
                for j in range(_DH2 // 16):
                    cs = pl.ds(j * 16, 16)
                    rows_c[r, cs] = rows_c[r, cs] * spl
        pltpu.sync_copy(rows_c, acc.at[didx_c], add=True)

    def chunk(k, _):
        @pl.when(k % 2 == 0)
        def _():
            process(k, rows0, ibuf0, didx0, semg0, semi0,
                    rows1, ibuf1, semg1, semi1)

        @pl.when(k % 2 == 1)
        def _():
            process(k, rows1, ibuf1, didx1, semg1, semi1,
                    rows0, ibuf0, semg0, semi0)
        return 0
    lax.fori_loop(0, _NCHUNK, chunk, 0)

    plsc.subcore_barrier()

    def oslice(t, _):
        idx = t * _NS + sid

        @pl.when(idx < nchunks)
        def _():
            pltpu.sync_copy(acc.at[pl.ds(idx * 16, 16), :], stage)
            pltpu.sync_copy(stage, out_hbm.at[cid, pl.ds(idx * 16, 16), :])
        return 0
    lax.fori_loop(0, nit, oslice, 0)


def _layer(h, src, dst, et, invc_flat, comp, bases, root, bias):
    nb = comp.shape[1]
    comp_p = jnp.pad(comp, ((0, 0), (0, 32 - nb)))
    bases_p = jnp.pad(bases.reshape(nb, -1), ((0, 32 - nb), (0, 0)))
    wflat = _mm_bias(comp_p, bases_p, jnp.zeros((1, _D * _D), jnp.float32),
                     _R, 2048)
    wcat = wflat.reshape(_R, _D, _D).transpose(1, 0, 2).reshape(_D, _R * _D)
    zcat = jnp.zeros((1, _R * _D), jnp.float32)
    hcat = _mm_bias(h, wcat, zcat, 1000, _R * _D).reshape(2 * _NR, _DH2)
    hroot = _mm_bias(h, root, bias[None, :], 1000, _D)
    parts = _edge_kernel(hcat, src, dst, et, invc_flat)
    return _combine(hroot, parts)


def kernel(x, edge_index, edge_type, lin_w, lin_b,
           comp1, bases1, root1, bias1, comp2, bases2, root2, bias2):
    src = edge_index[0]
    dst = edge_index[1]
    et = edge_type
    zc = jnp.zeros((_NR,), jnp.float32)

    h0 = _mm_bias(x, lin_w.T, lin_b[None, :], 1000, _D)

    cparts = _counts_kernel(dst, et, zc)
    invc_flat = _invc(cparts.reshape(_NC, 625, _D)).reshape(_NR)

    comp_s = jnp.stack([comp1, comp2])
    bases_s = jnp.stack([bases1, bases2])
    root_s = jnp.stack([root1, root2])
    bias_s = jnp.stack([bias1, bias2])

    def body(h, ws):
        comp, bases, root, bias = ws
        return _layer(h, src, dst, et, invc_flat, comp, bases, root, bias), 0

    h2, _ = lax.scan(body, h0, (comp_s, bases_s, root_s, bias_s))
    return h2


# speedup vs baseline: 36.8416x; 1.0074x over previous
import functools

import jax
import jax.numpy as jnp
from jax import lax
from jax.experimental import pallas as pl
from jax.experimental.pallas import tpu as pltpu
from jax.experimental.pallas import tpu_sc as plsc

_N = 10000
_E = 320000
_R = 8
_D = 128
_NR = _N * _R
_NC = 2
_NS = 16
_NW = _NC * _NS
_EPW = _E // _NW
_EPT = _E // _NS
_DH2 = _D // 2
_CH = 80
_NCHUNK = _EPT // _CH
_CCH = 2000
_CPW = _NR // _NS
_RPT = _N // _NS

_mesh = plsc.VectorSubcoreMesh(core_axis_name="c", subcore_axis_name="s")
_sc_params = pltpu.CompilerParams(needs_layout_passes=False,
                                  use_tc_tiling_on_sc=False)


def _mm_bias(a, b, bias, bm, bn):
    M, K = a.shape
    Nn = b.shape[1]

    def body(a_ref, b_ref, s_ref, o_ref):
        o_ref[...] = jnp.dot(a_ref[...], b_ref[...],
                             preferred_element_type=jnp.float32) + s_ref[...]

    return pl.pallas_call(
        body,
        grid=(M // bm, Nn // bn),
        in_specs=[
            pl.BlockSpec((bm, K), lambda i, j: (i, 0)),
            pl.BlockSpec((K, bn), lambda i, j: (0, j)),
            pl.BlockSpec((1, bn), lambda i, j: (0, j)),
        ],
        out_specs=pl.BlockSpec((bm, bn), lambda i, j: (i, j)),
        out_shape=jax.ShapeDtypeStruct((M, Nn), jnp.float32),
    )(a, b, bias)


def _combine(hroot, parts):
    M, Nn = hroot.shape

    def body(h_ref, p_ref, o_ref):
        o_ref[...] = h_ref[...] + jnp.concatenate(
            [p_ref[0], p_ref[1]], axis=1)

    bm = 1000
    return pl.pallas_call(
        body,
        grid=(M // bm,),
        in_specs=[
            pl.BlockSpec((bm, Nn), lambda i: (i, 0)),
            pl.BlockSpec((_NC, bm, Nn // 2), lambda i: (0, i, 0)),
        ],
        out_specs=pl.BlockSpec((bm, Nn), lambda i: (i, 0)),
        out_shape=jax.ShapeDtypeStruct((M, Nn), jnp.float32),
    )(hroot, parts)


def _invc(cparts):
    def body(c_ref, o_ref):
        o_ref[...] = 1.0 / jnp.maximum(c_ref[0] + c_ref[1], 1.0)

    return pl.pallas_call(
        body,
        out_shape=jax.ShapeDtypeStruct((625, 128), jnp.float32),
    )(cparts)


@functools.partial(
    pl.kernel,
    out_type=jax.ShapeDtypeStruct((_NC * _NR,), jnp.float32),
    mesh=_mesh,
    scratch_types=[
        pltpu.VMEM_SHARED((_NR,), jnp.float32),
        pltpu.VMEM((_CCH,), jnp.int32),
        pltpu.VMEM((_CCH,), jnp.int32),
        pltpu.VMEM((_CCH,), jnp.int32),
        pltpu.VMEM((_CCH,), jnp.float32),
        pltpu.VMEM((_CPW,), jnp.float32),
    ],
    compiler_params=_sc_params,
)
def _counts_kernel(dst_hbm, et_hbm, zero_hbm, out_hbm,
                   cacc, dbuf, ebuf, fbuf, obuf, stage):
    cid = lax.axis_index("c")
    sid = lax.axis_index("s")
    wid = sid * _NC + cid
    pltpu.sync_copy(zero_hbm.at[pl.ds(sid * _CPW, _CPW)], stage)
    pltpu.sync_copy(stage, cacc.at[pl.ds(sid * _CPW, _CPW)])
    def fill(g, _):
        obuf[pl.ds(g * 16, 16)] = jnp.ones((16,), jnp.float32)
        return 0
    lax.fori_loop(0, _CCH // 16, fill, 0)
    plsc.subcore_barrier()

    base = wid * _EPW

    def chunk(k, _):
        off = base + k * _CCH
        pltpu.sync_copy(dst_hbm.at[pl.ds(off, _CCH)], dbuf)
        pltpu.sync_copy(et_hbm.at[pl.ds(off, _CCH)], ebuf)

        def grp(g, _):
            sl = pl.ds(g * 16, 16)
            fbuf[sl] = dbuf[sl] * 8 + ebuf[sl]
            return 0
        lax.fori_loop(0, _CCH // 16, grp, 0)
        pltpu.sync_copy(obuf, cacc.at[fbuf], add=True)
        return 0
    lax.fori_loop(0, _EPW // _CCH, chunk, 0)

    plsc.subcore_barrier()
    pltpu.sync_copy(cacc.at[pl.ds(sid * _CPW, _CPW)], stage)
    pltpu.sync_copy(stage, out_hbm.at[pl.ds(cid * _NR + sid * _CPW, _CPW)])


@functools.partial(
    pl.kernel,
    out_type=jax.ShapeDtypeStruct((_NC, _N, _DH2), jnp.float32),
    mesh=_mesh,
    scratch_types=[
        pltpu.VMEM_SHARED((_N, _DH2), jnp.float32),
        pltpu.VMEM((_EPT,), jnp.int32),
        pltpu.VMEM((_EPT,), jnp.int32),
        pltpu.VMEM((_EPT,), jnp.int32),
        pltpu.VMEM((_CH, _DH2), jnp.float32),
        pltpu.VMEM((_CH, _DH2), jnp.float32),
        pltpu.VMEM((_CH,), jnp.float32),
        pltpu.VMEM((_CH,), jnp.float32),
        pltpu.VMEM((_CH,), jnp.int32),
        pltpu.VMEM((_CH,), jnp.int32),
        pltpu.VMEM((16, _DH2), jnp.float32),
        pltpu.SemaphoreType.DMA,
        pltpu.SemaphoreType.DMA,
        pltpu.SemaphoreType.DMA,
        pltpu.SemaphoreType.DMA,
        pltpu.SemaphoreType.DMA,
        pltpu.SemaphoreType.DMA,
    ],
    compiler_params=_sc_params,
)
def _edge_kernel(hcat_hbm, src_hbm, dst_hbm, et_hbm, invc_hbm,
                 out_hbm, acc, gbuf, fbuf, dbuf, rows0, rows1, ibuf0, ibuf1,
                 didx0, didx1, stage, semg0, semg1, semi0, semi1,
                 sems0, sems1):
    cid = lax.axis_index("c")
    sid = lax.axis_index("s")
    for a in range(16):
        for b in range(_DH2 // 16):
            stage[a, pl.ds(b * 16, 16)] = jnp.zeros((16,), jnp.float32)
    nchunks = _N // 16
    nit = (nchunks + _NS - 1) // _NS

    def zslice(t, _):
        idx = t * _NS + sid

        @pl.when(idx < nchunks)
        def _():
            pltpu.sync_copy(stage, acc.at[pl.ds(idx * 16, 16), :])
        return 0
    lax.fori_loop(0, nit, zslice, 0)
    plsc.subcore_barrier()

    base = sid * _EPT

    c1 = pltpu.async_copy(src_hbm.at[pl.ds(base, _EPT)], gbuf, semg0)
    c2 = pltpu.async_copy(et_hbm.at[pl.ds(base, _EPT)], fbuf, semg1)
    c3 = pltpu.async_copy(dst_hbm.at[pl.ds(base, _EPT)], dbuf, semi0)
    c1.wait()
    c2.wait()
    c3.wait()

    def idxgrp(g, _):
        sl = pl.ds(g * 16, 16)
        e16 = fbuf[sl]
        gbuf[sl] = (gbuf[sl] * 8 + e16) * 2 + cid
        fbuf[sl] = dbuf[sl] * 8 + e16
        return 0
    lax.fori_loop(0, _EPT // 16, idxgrp, 0)

    pltpu.async_copy(hcat_hbm.at[gbuf.at[pl.ds(0, _CH)]], rows0, semg0)
    pltpu.async_copy(invc_hbm.at[fbuf.at[pl.ds(0, _CH)]], ibuf0, semi0)

    def process(k, rows_c, ibuf_c, didx_c, semg_c, semi_c, sems_c,
                rows_n, ibuf_n, didx_n, semg_n, semi_n, sems_n):
        pltpu.make_async_copy(
            hcat_hbm.at[gbuf.at[pl.ds(0, _CH)]], rows_c, semg_c).wait()
        pltpu.make_async_copy(
            invc_hbm.at[fbuf.at[pl.ds(0, _CH)]], ibuf_c, semi_c).wait()

        @pl.when(k >= 1)
        def _():
            pltpu.make_async_copy(rows_n, acc.at[didx_n], sems_n).wait()

        @pl.when(k + 1 < _NCHUNK)
        def _():
            off = (k + 1) * _CH
            pltpu.async_copy(
                hcat_hbm.at[gbuf.at[pl.ds(off, _CH)]], rows_n, semg_n)
            pltpu.async_copy(
                invc_hbm.at[fbuf.at[pl.ds(off, _CH)]], ibuf_n, semi_n)
        off0 = k * _CH
        for g in range(_CH // 16):
            didx_c[pl.ds(g * 16, 16)] = dbuf[pl.ds(off0 + g * 16, 16)]
        for g in range(_CH // 16):
            v16 = ibuf_c[pl.ds(g * 16, 16)]
            for i in range(16):
                spl = jnp.broadcast_to(v16[i], (16,))
                r = g * 16 + i
                for j in range(_DH2 // 16):
                    cs = pl.ds(j * 16, 16)
                    rows_c[r, cs] = rows_c[r, cs] * spl
        pltpu.async_copy(rows_c, acc.at[didx_c], sems_c, add=True)

    def chunk(k, _):
        @pl.when(k % 2 == 0)
        def _():
            process(k, rows0, ibuf0, didx0, semg0, semi0, sems0,
                    rows1, ibuf1, didx1, semg1, semi1, sems1)

        @pl.when(k % 2 == 1)
        def _():
            process(k, rows1, ibuf1, didx1, semg1, semi1, sems1,
                    rows0, ibuf0, didx0, semg0, semi0, sems0)
        return 0
    lax.fori_loop(0, _NCHUNK, chunk, 0)
    pltpu.make_async_copy(rows1, acc.at[didx1], sems1).wait()

    plsc.subcore_barrier()

    def oslice(t, _):
        idx = t * _NS + sid

        @pl.when(idx < nchunks)
        def _():
            pltpu.sync_copy(acc.at[pl.ds(idx * 16, 16), :], stage)
            pltpu.sync_copy(stage, out_hbm.at[cid, pl.ds(idx * 16, 16), :])
        return 0
    lax.fori_loop(0, nit, oslice, 0)


def _basis(comp, bases):
    nb = comp.shape[1]
    comp_p = jnp.pad(comp, ((0, 0), (0, 32 - nb)))
    bases_p = jnp.pad(bases.reshape(nb, -1), ((0, 32 - nb), (0, 0)))
    wflat = _mm_bias(comp_p, bases_p, jnp.zeros((1, _D * _D), jnp.float32),
                     _R, 2048)
    return wflat.reshape(_R, _D, _D).transpose(1, 0, 2).reshape(_D, _R * _D)


def _layer(h, src, dst, et, invc_flat, wcat, root, bias):
    zcat = jnp.zeros((1, _R * _D), jnp.float32)
    hcat = _mm_bias(h, wcat, zcat, 1000, _R * _D).reshape(2 * _NR, _DH2)
    hroot = _mm_bias(h, root, bias[None, :], 1000, _D)
    parts = _edge_kernel(hcat, src, dst, et, invc_flat)
    return _combine(hroot, parts)


def kernel(x, edge_index, edge_type, lin_w, lin_b,
           comp1, bases1, root1, bias1, comp2, bases2, root2, bias2):
    src = edge_index[0]
    dst = edge_index[1]
    et = edge_type
    zc = jnp.zeros((_NR,), jnp.float32)

    h0 = _mm_bias(x, lin_w.T, lin_b[None, :], 1000, _D)

    cparts = _counts_kernel(dst, et, zc)
    invc_flat = _invc(cparts.reshape(_NC, 625, _D)).reshape(_NR)

    wcat_s = jnp.stack([_basis(comp1, bases1), _basis(comp2, bases2)])
    root_s = jnp.stack([root1, root2])
    bias_s = jnp.stack([bias1, bias2])

    def body(h, ws):
        wcat, root, bias = ws
        return _layer(h, src, dst, et, invc_flat, wcat, root, bias), 0

    h2, _ = lax.scan(body, h0, (wcat_s, root_s, bias_s))
    return h2


# speedup vs baseline: 36.9208x; 1.0022x over previous
import functools

import jax
import jax.numpy as jnp
from jax import lax
from jax.experimental import pallas as pl
from jax.experimental.pallas import tpu as pltpu
from jax.experimental.pallas import tpu_sc as plsc

_N = 10000
_E = 320000
_R = 8
_D = 128
_NR = _N * _R
_NC = 2
_NS = 16
_NW = _NC * _NS
_EPW = _E // _NW
_EPT = _E // _NS
_DH2 = _D // 2
_CH = 80
_NCHUNK = _EPT // _CH
_CCH = 2000
_CPW = _NR // _NS
_RPT = _N // _NS

_mesh = plsc.VectorSubcoreMesh(core_axis_name="c", subcore_axis_name="s")
_sc_params = pltpu.CompilerParams(needs_layout_passes=False,
                                  use_tc_tiling_on_sc=False)


def _mm_bias(a, b, bias, bm, bn):
    M, K = a.shape
    Nn = b.shape[1]

    def body(a_ref, b_ref, s_ref, o_ref):
        o_ref[...] = jnp.dot(a_ref[...], b_ref[...],
                             preferred_element_type=jnp.float32) + s_ref[...]

    return pl.pallas_call(
        body,
        grid=(M // bm, Nn // bn),
        in_specs=[
            pl.BlockSpec((bm, K), lambda i, j: (i, 0)),
            pl.BlockSpec((K, bn), lambda i, j: (0, j)),
            pl.BlockSpec((1, bn), lambda i, j: (0, j)),
        ],
        out_specs=pl.BlockSpec((bm, bn), lambda i, j: (i, j)),
        out_shape=jax.ShapeDtypeStruct((M, Nn), jnp.float32),
    )(a, b, bias)


def _combine(hroot, parts):
    M, Nn = hroot.shape

    def body(h_ref, p_ref, o_ref):
        o_ref[...] = h_ref[...] + jnp.concatenate(
            [p_ref[0], p_ref[1]], axis=1)

    bm = 1000
    return pl.pallas_call(
        body,
        grid=(M // bm,),
        in_specs=[
            pl.BlockSpec((bm, Nn), lambda i: (i, 0)),
            pl.BlockSpec((_NC, bm, Nn // 2), lambda i: (0, i, 0)),
        ],
        out_specs=pl.BlockSpec((bm, Nn), lambda i: (i, 0)),
        out_shape=jax.ShapeDtypeStruct((M, Nn), jnp.float32),
    )(hroot, parts)


def _invc(cparts):
    def body(c_ref, o_ref):
        o_ref[...] = 1.0 / jnp.maximum(c_ref[0] + c_ref[1], 1.0)

    return pl.pallas_call(
        body,
        out_shape=jax.ShapeDtypeStruct((625, 128), jnp.float32),
    )(cparts)


@functools.partial(
    pl.kernel,
    out_type=jax.ShapeDtypeStruct((_NC * _NR,), jnp.float32),
    mesh=_mesh,
    scratch_types=[
        pltpu.VMEM_SHARED((_NR,), jnp.float32),
        pltpu.VMEM((_CCH,), jnp.int32),
        pltpu.VMEM((_CCH,), jnp.int32),
        pltpu.VMEM((_CCH,), jnp.int32),
        pltpu.VMEM((_CCH,), jnp.float32),
        pltpu.VMEM((_CPW,), jnp.float32),
    ],
    compiler_params=_sc_params,
)
def _counts_kernel(dst_hbm, et_hbm, zero_hbm, out_hbm,
                   cacc, dbuf, ebuf, fbuf, obuf, stage):
    cid = lax.axis_index("c")
    sid = lax.axis_index("s")
    wid = sid * _NC + cid
    pltpu.sync_copy(zero_hbm.at[pl.ds(sid * _CPW, _CPW)], stage)
    pltpu.sync_copy(stage, cacc.at[pl.ds(sid * _CPW, _CPW)])
    def fill(g, _):
        obuf[pl.ds(g * 16, 16)] = jnp.ones((16,), jnp.float32)
        return 0
    lax.fori_loop(0, _CCH // 16, fill, 0)
    plsc.subcore_barrier()

    base = wid * _EPW

    def chunk(k, _):
        off = base + k * _CCH
        pltpu.sync_copy(dst_hbm.at[pl.ds(off, _CCH)], dbuf)
        pltpu.sync_copy(et_hbm.at[pl.ds(off, _CCH)], ebuf)

        def grp(g, _):
            sl = pl.ds(g * 16, 16)
            fbuf[sl] = dbuf[sl] * 8 + ebuf[sl]
            return 0
        lax.fori_loop(0, _CCH // 16, grp, 0)
        pltpu.sync_copy(obuf, cacc.at[fbuf], add=True)
        return 0
    lax.fori_loop(0, _EPW // _CCH, chunk, 0)

    plsc.subcore_barrier()
    pltpu.sync_copy(cacc.at[pl.ds(sid * _CPW, _CPW)], stage)
    pltpu.sync_copy(stage, out_hbm.at[pl.ds(cid * _NR + sid * _CPW, _CPW)])


@functools.partial(
    pl.kernel,
    out_type=jax.ShapeDtypeStruct((_NC, _N, _DH2), jnp.float32),
    mesh=_mesh,
    scratch_types=[
        pltpu.VMEM_SHARED((_N, _DH2), jnp.float32),
        pltpu.VMEM((_EPT,), jnp.int32),
        pltpu.VMEM((_EPT,), jnp.int32),
        pltpu.VMEM((_EPT,), jnp.int32),
        pltpu.VMEM((_CH, _DH2), jnp.float32),
        pltpu.VMEM((_CH, _DH2), jnp.float32),
        pltpu.VMEM((_CH,), jnp.float32),
        pltpu.VMEM((_CH,), jnp.float32),
        pltpu.VMEM((_CH,), jnp.int32),
        pltpu.VMEM((_CH,), jnp.int32),
        pltpu.VMEM((16, _DH2), jnp.float32),
        pltpu.SemaphoreType.DMA,
        pltpu.SemaphoreType.DMA,
        pltpu.SemaphoreType.DMA,
        pltpu.SemaphoreType.DMA,
        pltpu.SemaphoreType.DMA,
        pltpu.SemaphoreType.DMA,
    ],
    compiler_params=_sc_params,
)
def _edge_kernel(hcat_hbm, src_hbm, dst_hbm, et_hbm, invc_hbm,
                 out_hbm, acc, gbuf, fbuf, dbuf, rows0, rows1, ibuf0, ibuf1,
                 didx0, didx1, stage, semg0, semg1, semi0, semi1,
                 sems0, sems1):
    cid = lax.axis_index("c")
    sid = lax.axis_index("s")
    for a in range(16):
        for b in range(_DH2 // 16):
            stage[a, pl.ds(b * 16, 16)] = jnp.zeros((16,), jnp.float32)
    nchunks = _N // 16
    nit = (nchunks + _NS - 1) // _NS

    def zslice(t, _):
        idx = t * _NS + sid

        @pl.when(idx < nchunks)
        def _():
            pltpu.sync_copy(stage, acc.at[pl.ds(idx * 16, 16), :])
        return 0
    lax.fori_loop(0, nit, zslice, 0)
    plsc.subcore_barrier()

    base = sid * _EPT

    c1 = pltpu.async_copy(src_hbm.at[pl.ds(base, _EPT)], gbuf, semg0)
    c2 = pltpu.async_copy(et_hbm.at[pl.ds(base, _EPT)], fbuf, semg1)
    c3 = pltpu.async_copy(dst_hbm.at[pl.ds(base, _EPT)], dbuf, semi0)
    c1.wait()
    c2.wait()
    c3.wait()

    def idxgrp(g, _):
        sl = pl.ds(g * 16, 16)
        e16 = fbuf[sl]
        gbuf[sl] = (gbuf[sl] * 8 + e16) * 2 + cid
        fbuf[sl] = dbuf[sl] * 8 + e16
        return 0
    lax.fori_loop(0, _EPT // 16, idxgrp, 0)

    pltpu.async_copy(hcat_hbm.at[gbuf.at[pl.ds(0, _CH // 2)]],
                     rows0.at[pl.ds(0, _CH // 2)], semg0)
    pltpu.async_copy(hcat_hbm.at[gbuf.at[pl.ds(_CH // 2, _CH // 2)]],
                     rows0.at[pl.ds(_CH // 2, _CH // 2)], semi0)
    pltpu.async_copy(invc_hbm.at[fbuf.at[pl.ds(0, _CH)]], ibuf0, semg0)

    def process(k, rows_c, ibuf_c, didx_c, semg_c, semi_c, sems_c,
                rows_n, ibuf_n, didx_n, semg_n, semi_n, sems_n):
        h2 = _CH // 2
        pltpu.make_async_copy(
            hcat_hbm.at[gbuf.at[pl.ds(0, h2)]],
            rows_c.at[pl.ds(0, h2)], semg_c).wait()
        pltpu.make_async_copy(
            hcat_hbm.at[gbuf.at[pl.ds(0, h2)]],
            rows_c.at[pl.ds(h2, h2)], semi_c).wait()
        pltpu.make_async_copy(
            invc_hbm.at[fbuf.at[pl.ds(0, _CH)]], ibuf_c, semg_c).wait()

        @pl.when(k >= 1)
        def _():
            pltpu.make_async_copy(rows_n, acc.at[didx_n], sems_n).wait()

        @pl.when(k + 1 < _NCHUNK)
        def _():
            off = (k + 1) * _CH
            h2n = _CH // 2
            pltpu.async_copy(
                hcat_hbm.at[gbuf.at[pl.ds(off, h2n)]],
                rows_n.at[pl.ds(0, h2n)], semg_n)
            pltpu.async_copy(
                hcat_hbm.at[gbuf.at[pl.ds(off + h2n, h2n)]],
                rows_n.at[pl.ds(h2n, h2n)], semi_n)
            pltpu.async_copy(
                invc_hbm.at[fbuf.at[pl.ds(off, _CH)]], ibuf_n, semg_n)
        off0 = k * _CH
        for g in range(_CH // 16):
            didx_c[pl.ds(g * 16, 16)] = dbuf[pl.ds(off0 + g * 16, 16)]
        for g in range(_CH // 16):
            v16 = ibuf_c[pl.ds(g * 16, 16)]
            for i in range(16):
                spl = jnp.broadcast_to(v16[i], (16,))
                r = g * 16 + i
                for j in range(_DH2 // 16):
                    cs = pl.ds(j * 16, 16)
                    rows_c[r, cs] = rows_c[r, cs] * spl
        pltpu.async_copy(rows_c, acc.at[didx_c], sems_c, add=True)

    def chunk(k, _):
        @pl.when(k % 2 == 0)
        def _():
            process(k, rows0, ibuf0, didx0, semg0, semi0, sems0,
                    rows1, ibuf1, didx1, semg1, semi1, sems1)

        @pl.when(k % 2 == 1)
        def _():
            process(k, rows1, ibuf1, didx1, semg1, semi1, sems1,
                    rows0, ibuf0, didx0, semg0, semi0, sems0)
        return 0
    lax.fori_loop(0, _NCHUNK, chunk, 0)
    pltpu.make_async_copy(rows1, acc.at[didx1], sems1).wait()

    plsc.subcore_barrier()

    def oslice(t, _):
        idx = t * _NS + sid

        @pl.when(idx < nchunks)
        def _():
            pltpu.sync_copy(acc.at[pl.ds(idx * 16, 16), :], stage)
            pltpu.sync_copy(stage, out_hbm.at[cid, pl.ds(idx * 16, 16), :])
        return 0
    lax.fori_loop(0, nit, oslice, 0)


def _basis(comp, bases):
    nb = comp.shape[1]
    comp_p = jnp.pad(comp, ((0, 0), (0, 32 - nb)))
    bases_p = jnp.pad(bases.reshape(nb, -1), ((0, 32 - nb), (0, 0)))
    wflat = _mm_bias(comp_p, bases_p, jnp.zeros((1, _D * _D), jnp.float32),
                     _R, 2048)
    return wflat.reshape(_R, _D, _D).transpose(1, 0, 2).reshape(_D, _R * _D)


def _layer(h, src, dst, et, invc_flat, wcat, root, bias):
    zcat = jnp.zeros((1, _R * _D), jnp.float32)
    hcat = _mm_bias(h, wcat, zcat, 1000, _R * _D).reshape(2 * _NR, _DH2)
    hroot = _mm_bias(h, root, bias[None, :], 1000, _D)
    parts = _edge_kernel(hcat, src, dst, et, invc_flat)
    return _combine(hroot, parts)


def kernel(x, edge_index, edge_type, lin_w, lin_b,
           comp1, bases1, root1, bias1, comp2, bases2, root2, bias2):
    src = edge_index[0]
    dst = edge_index[1]
    et = edge_type
    zc = jnp.zeros((_NR,), jnp.float32)

    h0 = _mm_bias(x, lin_w.T, lin_b[None, :], 1000, _D)

    cparts = _counts_kernel(dst, et, zc)
    invc_flat = _invc(cparts.reshape(_NC, 625, _D)).reshape(_NR)

    wcat_s = jnp.stack([_basis(comp1, bases1), _basis(comp2, bases2)])
    root_s = jnp.stack([root1, root2])
    bias_s = jnp.stack([bias1, bias2])

    def body(h, ws):
        wcat, root, bias = ws
        return _layer(h, src, dst, et, invc_flat, wcat, root, bias), 0

    h2, _ = lax.scan(body, h0, (wcat_s, root_s, bias_s))
    return h2
